# Initial kernel scaffold; baseline (speedup 1.0000x reference)
import jax
import jax.numpy as jnp
from jax import lax
from jax.experimental import pallas as pl
from jax.experimental.pallas import tpu as pltpu
from jax.experimental.pallas import tpu_sc as plsc

_NUM_IDS = 16384
_EMBED_DIM = 32
_NNZ = 819200
_C = 128
_K = 20
_NW = 32
_NNZ_PER_TILE = _NNZ // _NW
_NB = _NNZ_PER_TILE // (_K * _C)
_ROWS_PER_TILE = _NUM_IDS // 16


def _sc_body(col2, row2, data2, emb, partial_out, colb, rowb, datab, rbuf,
             accum, gsem):
    c = lax.axis_index("c")
    s = lax.axis_index("s")
    w = c * 16 + s

    zero = jnp.zeros((16,), jnp.float32)

    def _zr(i, _):
        rbuf[i, pl.ds(0, 16)] = zero
        rbuf[i, pl.ds(16, 16)] = zero
        return 0

    lax.fori_loop(0, _C, _zr, 0)

    def _zc(k, _):
        pltpu.sync_copy(rbuf, accum.at[pl.ds(s * _ROWS_PER_TILE + k * _C, _C)])
        return 0

    lax.fori_loop(0, _ROWS_PER_TILE // _C, _zc, 0)
    plsc.subcore_barrier()

    iota16 = lax.iota(jnp.int32, 16)

    def _block(b, _):
        blk0 = w * (_NNZ_PER_TILE // _C) + b * _K
        pltpu.sync_copy(col2.at[pl.ds(blk0, _K)], colb)
        pltpu.sync_copy(row2.at[pl.ds(blk0, _K)], rowb)
        pltpu.sync_copy(data2.at[pl.ds(blk0, _K)], datab)

        def _sub(j, _):
            pltpu.async_copy(emb.at[colb.at[j]], rbuf, gsem).wait()

            def _grp(g, _):
                dvec = datab[j, pl.ds(g * 16, 16)]
                rid = g * 16 + iota16
                for d in range(_EMBED_DIM):
                    cid = jnp.full((16,), d, jnp.int32)
                    v = plsc.load_gather(rbuf, [rid, cid])
                    plsc.store_scatter(rbuf, [rid, cid], v * dvec)
                return 0

            lax.fori_loop(0, _C // 16, _grp, 0)
            pltpu.sync_copy(rbuf, accum.at[rowb.at[j]], add=True)
            return 0

        lax.fori_loop(0, _K, _sub, 0)
        return 0

    lax.fori_loop(0, _NB, _block, 0)
    plsc.subcore_barrier()
    pltpu.sync_copy(accum.at[pl.ds(s * _ROWS_PER_TILE, _ROWS_PER_TILE)],
                    partial_out.at[c, pl.ds(s * _ROWS_PER_TILE, _ROWS_PER_TILE)])


def _sc_call(col2, row2, data2, emb):
    mesh = plsc.VectorSubcoreMesh(core_axis_name="c", subcore_axis_name="s")
    return pl.kernel(
        _sc_body,
        out_type=jax.ShapeDtypeStruct((2, _NUM_IDS, _EMBED_DIM), jnp.float32),
        mesh=mesh,
        scratch_types=[
            pltpu.VMEM((_K, _C), jnp.int32),
            pltpu.VMEM((_K, _C), jnp.int32),
            pltpu.VMEM((_K, _C), jnp.float32),
            pltpu.VMEM((_C, _EMBED_DIM), jnp.float32),
            pltpu.VMEM_SHARED((_NUM_IDS, _EMBED_DIM), jnp.float32),
            pltpu.SemaphoreType.DMA,
        ],
    )(col2, row2, data2, emb)


def _add_body(a_ref, b_ref, o_ref):
    o_ref[...] = a_ref[...] + b_ref[...]


def _combine(pa, pb):
    return pl.pallas_call(
        _add_body,
        out_shape=jax.ShapeDtypeStruct((_NUM_IDS, _EMBED_DIM), jnp.float32),
        grid=(8,),
        in_specs=[
            pl.BlockSpec((_NUM_IDS // 8, _EMBED_DIM), lambda i: (i, 0)),
            pl.BlockSpec((_NUM_IDS // 8, _EMBED_DIM), lambda i: (i, 0)),
        ],
        out_specs=pl.BlockSpec((_NUM_IDS // 8, _EMBED_DIM), lambda i: (i, 0)),
    )(pa, pb)


def kernel(row_idx, col_idx, data_tensor, num_ids, embeddings):
    del num_ids
    row2 = row_idx.reshape(_NNZ // _C, _C)
    col2 = col_idx.reshape(_NNZ // _C, _C)
    data2 = data_tensor.reshape(_NNZ // _C, _C)
    partials = _sc_call(col2, row2, data2, embeddings)
    return _combine(partials[0], partials[1])


# speedup vs baseline: 4.2059x; 4.2059x over previous
import jax
import jax.numpy as jnp
from jax import lax
from jax.experimental import pallas as pl
from jax.experimental.pallas import tpu as pltpu
from jax.experimental.pallas import tpu_sc as plsc

_NUM_IDS = 16384
_EMBED_DIM = 32
_NNZ = 819200
_C = 128
_K = 40
_NW = 32
_NNZ_PER_TILE = _NNZ // _NW
_NB = _NNZ_PER_TILE // (_K * _C)
_ROWS_PER_TILE = _NUM_IDS // 16


def _sc_body(col2, row2, data2, emb, partial_out, colb, rowb, datab, rbuf,
             accum, gsem):
    c = lax.axis_index("c")
    s = lax.axis_index("s")
    w = c * 16 + s

    zero = jnp.zeros((16,), jnp.float32)

    def _zr(i, _):
        rbuf[i, pl.ds(0, 16)] = zero
        rbuf[i, pl.ds(16, 16)] = zero
        return 0

    lax.fori_loop(0, _C, _zr, 0)

    def _zc(k, _):
        pltpu.sync_copy(rbuf, accum.at[pl.ds(s * _ROWS_PER_TILE + k * _C, _C)])
        return 0

    lax.fori_loop(0, _ROWS_PER_TILE // _C, _zc, 0)
    plsc.subcore_barrier()

    iota16 = lax.iota(jnp.int32, 16)

    def _block(b, _):
        blk0 = w * (_NNZ_PER_TILE // _C) + b * _K
        pltpu.sync_copy(col2.at[pl.ds(blk0, _K)], colb)
        pltpu.sync_copy(row2.at[pl.ds(blk0, _K)], rowb)
        pltpu.sync_copy(data2.at[pl.ds(blk0, _K)], datab)

        def _sub(j, _):
            pltpu.async_copy(emb.at[colb.at[j]], rbuf, gsem).wait()

            def _grp(g, _):
                dvec = datab[j, pl.ds(g * 16, 16)]
                for i in range(16):
                    spl = jnp.take_along_axis(
                        dvec, jnp.full((16,), i, jnp.int32), axis=0)
                    r = g * 16 + i
                    rbuf[r, pl.ds(0, 16)] = rbuf[r, pl.ds(0, 16)] * spl
                    rbuf[r, pl.ds(16, 16)] = rbuf[r, pl.ds(16, 16)] * spl
                return 0

            lax.fori_loop(0, _C // 16, _grp, 0)
            pltpu.sync_copy(rbuf, accum.at[rowb.at[j]], add=True)
            return 0

        lax.fori_loop(0, _K, _sub, 0)
        return 0

    lax.fori_loop(0, _NB, _block, 0)
    plsc.subcore_barrier()
    pltpu.sync_copy(accum.at[pl.ds(s * _ROWS_PER_TILE, _ROWS_PER_TILE)],
                    partial_out.at[c, pl.ds(s * _ROWS_PER_TILE, _ROWS_PER_TILE)])


def _sc_call(col2, row2, data2, emb):
    mesh = plsc.VectorSubcoreMesh(core_axis_name="c", subcore_axis_name="s")
    return pl.kernel(
        _sc_body,
        out_type=jax.ShapeDtypeStruct((2, _NUM_IDS, _EMBED_DIM), jnp.float32),
        mesh=mesh,
        compiler_params=pltpu.CompilerParams(use_tc_tiling_on_sc=False),
        scratch_types=[
            pltpu.VMEM((_K, _C), jnp.int32),
            pltpu.VMEM((_K, _C), jnp.int32),
            pltpu.VMEM((_K, _C), jnp.float32),
            pltpu.VMEM((_C, _EMBED_DIM), jnp.float32),
            pltpu.VMEM_SHARED((_NUM_IDS, _EMBED_DIM), jnp.float32),
            pltpu.SemaphoreType.DMA,
        ],
    )(col2, row2, data2, emb)


def _add_body(a_ref, b_ref, o_ref):
    o_ref[...] = a_ref[...] + b_ref[...]


def _combine(pa, pb):
    return pl.pallas_call(
        _add_body,
        out_shape=jax.ShapeDtypeStruct((_NUM_IDS, _EMBED_DIM), jnp.float32),
        grid=(8,),
        in_specs=[
            pl.BlockSpec((_NUM_IDS // 8, _EMBED_DIM), lambda i: (i, 0)),
            pl.BlockSpec((_NUM_IDS // 8, _EMBED_DIM), lambda i: (i, 0)),
        ],
        out_specs=pl.BlockSpec((_NUM_IDS // 8, _EMBED_DIM), lambda i: (i, 0)),
    )(pa, pb)


def kernel(row_idx, col_idx, data_tensor, num_ids, embeddings):
    del num_ids
    row2 = row_idx.reshape(_NNZ // _C, _C)
    col2 = col_idx.reshape(_NNZ // _C, _C)
    data2 = data_tensor.reshape(_NNZ // _C, _C)
    partials = _sc_call(col2, row2, data2, embeddings)
    return _combine(partials[0], partials[1])


# speedup vs baseline: 5.0938x; 1.2111x over previous
import jax
import jax.numpy as jnp
from jax import lax
from jax.experimental import pallas as pl
from jax.experimental.pallas import tpu as pltpu
from jax.experimental.pallas import tpu_sc as plsc

_NUM_IDS = 16384
_EMBED_DIM = 32
_NNZ = 819200
_C = 128
_NW = 32
_NT = _NNZ // _NW
_NCH = _NT // _C
_NSLOT = 4
_NQ = _NCH // _NSLOT
_ROWS_PER_TILE = _NUM_IDS // 16


def _sc_body(col_h, row_h, data_h, emb, partial_out,
             colb, rowb2, datab, rb0, rb1, rb2, rb3, accum,
             g0, g1, g2, g3, s0, s1, s2, s3):
    c = lax.axis_index("c")
    s = lax.axis_index("s")
    w = c * 16 + s
    rbufs = (rb0, rb1, rb2, rb3)
    gsems = (g0, g1, g2, g3)
    ssems = (s0, s1, s2, s3)

    base = w * _NT
    pltpu.sync_copy(row_h.at[pl.ds(base, _NT)], colb)

    def _xp(i, _):
        v = colb[pl.ds(i * 16, 16)]
        rowb2[i // 8, pl.ds((i % 8) * 16, 16)] = v
        return 0

    lax.fori_loop(0, _NT // 16, _xp, 0)
    pltpu.sync_copy(col_h.at[pl.ds(base, _NT)], colb)
    pltpu.sync_copy(data_h.at[pl.ds(base, _NT)], datab)

    zero = jnp.zeros((16,), jnp.float32)

    def _zr(i, _):
        rb0[i, pl.ds(0, 16)] = zero
        rb0[i, pl.ds(16, 16)] = zero
        return 0

    lax.fori_loop(0, _C, _zr, 0)

    def _zc(k, _):
        pltpu.sync_copy(rb0, accum.at[pl.ds(s * _ROWS_PER_TILE + k * _C, _C)])
        return 0

    lax.fori_loop(0, _ROWS_PER_TILE // _C, _zc, 0)
    plsc.subcore_barrier()

    def _gather_start(j, b):
        pltpu.async_copy(emb.at[colb.at[pl.ds(j * _C, _C)]], rbufs[b],
                         gsems[b])

    def _gather_wait(b):
        pltpu.make_async_copy(emb.at[colb.at[pl.ds(0, _C)]], rbufs[b],
                              gsems[b]).wait()

    def _scatter_start(j, b):
        pltpu.async_copy(rbufs[b], accum.at[rowb2.at[j]], ssems[b], add=True)

    def _scatter_wait(b):
        pltpu.make_async_copy(rbufs[b], accum.at[rowb2.at[0]],
                              ssems[b]).wait()

    def _compute(j, b):
        rb = rbufs[b]

        def _grp(g, _):
            dvec = datab[pl.ds(j * _C + g * 16, 16)]
            for i in range(16):
                spl = jnp.take_along_axis(
                    dvec, jnp.full((16,), i, jnp.int32), axis=0)
                r = g * 16 + i
                rb[r, pl.ds(0, 16)] = rb[r, pl.ds(0, 16)] * spl
                rb[r, pl.ds(16, 16)] = rb[r, pl.ds(16, 16)] * spl
            return 0

        lax.fori_loop(0, _C // 16, _grp, 0)

    for b in range(_NSLOT):
        _gather_start(b, b)

    def _quad(q, _):
        for b in range(_NSLOT):
            j = q * _NSLOT + b
            _gather_wait(b)
            _compute(j, b)
            _scatter_start(j, b)
        for b in range(_NSLOT):
            jn = (q + 1) * _NSLOT + b
            _scatter_wait(b)
            _gather_start(jn, b)
        return 0

    lax.fori_loop(0, _NQ - 1, _quad, 0)
    for b in range(_NSLOT):
        j = (_NQ - 1) * _NSLOT + b
        _gather_wait(b)
        _compute(j, b)
        _scatter_start(j, b)
    for b in range(_NSLOT):
        _scatter_wait(b)

    plsc.subcore_barrier()
    pltpu.sync_copy(accum.at[pl.ds(s * _ROWS_PER_TILE, _ROWS_PER_TILE)],
                    partial_out.at[c, pl.ds(s * _ROWS_PER_TILE, _ROWS_PER_TILE)])


def _sc_call(col_idx, row_idx, data_tensor, emb):
    mesh = plsc.VectorSubcoreMesh(core_axis_name="c", subcore_axis_name="s")
    return pl.kernel(
        _sc_body,
        out_type=jax.ShapeDtypeStruct((2, _NUM_IDS, _EMBED_DIM), jnp.float32),
        mesh=mesh,
        compiler_params=pltpu.CompilerParams(use_tc_tiling_on_sc=False),
        scratch_types=[
            pltpu.VMEM((_NT,), jnp.int32),
            pltpu.VMEM((_NCH, _C), jnp.int32),
            pltpu.VMEM((_NT,), jnp.float32),
            pltpu.VMEM((_C, _EMBED_DIM), jnp.float32),
            pltpu.VMEM((_C, _EMBED_DIM), jnp.float32),
            pltpu.VMEM((_C, _EMBED_DIM), jnp.float32),
            pltpu.VMEM((_C, _EMBED_DIM), jnp.float32),
            pltpu.VMEM_SHARED((_NUM_IDS, _EMBED_DIM), jnp.float32),
            pltpu.SemaphoreType.DMA,
            pltpu.SemaphoreType.DMA,
            pltpu.SemaphoreType.DMA,
            pltpu.SemaphoreType.DMA,
            pltpu.SemaphoreType.DMA,
            pltpu.SemaphoreType.DMA,
            pltpu.SemaphoreType.DMA,
            pltpu.SemaphoreType.DMA,
        ],
    )(col_idx, row_idx, data_tensor, emb)


def _add_body(a_ref, b_ref, o_ref):
    o_ref[...] = a_ref[...] + b_ref[...]


def _combine(pa, pb):
    return pl.pallas_call(
        _add_body,
        out_shape=jax.ShapeDtypeStruct((_NUM_IDS, _EMBED_DIM), jnp.float32),
        grid=(8,),
        in_specs=[
            pl.BlockSpec((_NUM_IDS // 8, _EMBED_DIM), lambda i: (i, 0)),
            pl.BlockSpec((_NUM_IDS // 8, _EMBED_DIM), lambda i: (i, 0)),
        ],
        out_specs=pl.BlockSpec((_NUM_IDS // 8, _EMBED_DIM), lambda i: (i, 0)),
    )(pa, pb)


def kernel(row_idx, col_idx, data_tensor, num_ids, embeddings):
    del num_ids
    partials = _sc_call(col_idx, row_idx, data_tensor, embeddings)
    return _combine(partials[0], partials[1])
